# Initial kernel scaffold; baseline (speedup 1.0000x reference)
import jax
import jax.numpy as jnp
from jax.experimental import pallas as pl


def kernel(x, table):
    raise NotImplementedError("write your pallas kernel here")


# speedup vs baseline: 2.4948x; 2.4948x over previous
import functools

import jax
import jax.numpy as jnp
from jax import lax
from jax.experimental import pallas as pl
from jax.experimental.pallas import tpu as pltpu
from jax.experimental.pallas import tpu_sc as plsc

D_MODEL = 2048
N_ROWS = 4 * 8192
_L = 16

_NC = 2
_NS = 16
_NW = _NC * _NS
_BPW = N_ROWS // _NW
_C = 16
_NPAIR = _BPW // (2 * _C)

_mesh = plsc.VectorSubcoreMesh(core_axis_name="c", subcore_axis_name="s")


@functools.partial(
    pl.kernel,
    out_type=jax.ShapeDtypeStruct((N_ROWS, D_MODEL), jnp.float32),
    mesh=_mesh,
    scratch_types=[
        pltpu.VMEM((D_MODEL,), jnp.float32),
        pltpu.VMEM((D_MODEL,), jnp.float32),
        pltpu.VMEM((_BPW // 8, 128), jnp.int32),
        pltpu.VMEM((_C, D_MODEL), jnp.float32),
        pltpu.VMEM((_C, D_MODEL), jnp.float32),
        pltpu.SemaphoreType.DMA,
        pltpu.SemaphoreType.DMA,
    ],
)
def _embed_sc(xb_hbm, tab_hbm, out_hbm, row0_v, row1_v, idx_v, buf0, buf1,
              sem0, sem1):
    wid = lax.axis_index("s") * _NC + lax.axis_index("c")
    base = wid * _BPW
    pltpu.sync_copy(tab_hbm.at[0], row0_v)
    pltpu.sync_copy(tab_hbm.at[1], row1_v)
    pltpu.sync_copy(
        xb_hbm.at[pl.ds(pl.multiple_of(base // 8, 8), _BPW // 8)], idx_v)

    def build(cb, buf):
        cb8 = cb // 8
        preds = [
            idx_v[cb8 + (i // 8), pl.ds((i % 8) * _L, _L)] != 0
            for i in range(_C)
        ]

        def dbody(d, carry):
            off = pl.multiple_of(d * _L, _L)
            r0 = row0_v[pl.ds(off, _L)]
            r1 = row1_v[pl.ds(off, _L)]
            for i in range(_C):
                buf[i, pl.ds(off, _L)] = jnp.where(preds[i], r1, r0)
            return carry

        lax.fori_loop(0, D_MODEL // _L, dbody, 0, unroll=2)

    def fire(cb, buf, sem):
        gbase = pl.multiple_of(base + cb, 8)
        pltpu.async_copy(buf, out_hbm.at[pl.ds(gbase, _C)], sem)

    def drain(buf, sem):
        pltpu.make_async_copy(buf, out_hbm.at[pl.ds(base, _C)], sem).wait()

    build(0, buf0)
    fire(0, buf0, sem0)
    build(_C, buf1)
    fire(_C, buf1, sem1)

    def pair(j, carry):
        cb = pl.multiple_of(j * (2 * _C), _C)
        drain(buf0, sem0)
        build(cb, buf0)
        fire(cb, buf0, sem0)
        drain(buf1, sem1)
        build(cb + _C, buf1)
        fire(cb + _C, buf1, sem1)
        return carry

    lax.fori_loop(1, _NPAIR, pair, 0)
    drain(buf0, sem0)
    drain(buf1, sem1)


def kernel(x, table):
    xf = x.reshape(-1).astype(jnp.int32)
    xb = jnp.broadcast_to(xf[:, None], (N_ROWS, _L)).reshape(N_ROWS // 8, 128)
    out = _embed_sc(xb, table)
    return out.reshape(x.shape[0], x.shape[1], D_MODEL)
